# baseline (speedup 1.0000x reference)
import functools

import jax
import jax.numpy as jnp
from jax import lax
from jax.experimental import pallas as pl
from jax.experimental.pallas import tpu as pltpu
from jax.experimental.pallas import tpu_sc as plsc

MARGIN_MIN = 0.2
MARGIN_MAX = 0.5
B = 4096
D = 256
BLK = 128
NB = B // BLK
CH = 128
NCH = B // CH


def _mine_body(t_ref, v_ref, u_ref, tri_ref, out_ref):
    i = pl.program_id(0)
    t = t_ref[...]
    v = v_ref[...]
    s = lax.dot_general(
        t, v, (((1,), (1,)), ((), ())),
        precision=lax.Precision.HIGHEST,
        preferred_element_type=jnp.float32)
    col = lax.broadcasted_iota(jnp.int32, (BLK, B), 1)
    row = lax.broadcasted_iota(jnp.int32, (BLK, B), 0) + i * BLK
    is_diag = col == row
    colf = col.astype(jnp.float32)
    diag = jnp.sum(jnp.where(is_diag, s, 0.0), axis=1, keepdims=True)
    band = ((s > diag - MARGIN_MAX) & (s < diag - MARGIN_MIN)
            & jnp.logical_not(is_diag))
    bf = band.astype(jnp.float32)
    count = jnp.sum(bf, axis=1, keepdims=True)
    u = u_ref[...]
    k1 = jnp.floor(u * jnp.maximum(count, 1.0)) + 1.0
    tri = tri_ref[...]
    pref = jnp.zeros((BLK, 1), jnp.float32)
    cand = jnp.zeros((BLK, 1), jnp.float32)
    for m in range(NCH):
        bm = bf[:, m * CH:(m + 1) * CH]
        lm = lax.dot_general(
            bm, tri, (((1,), (0,)), ((), ())),
            preferred_element_type=jnp.float32)
        hit = ((pref + lm) == k1) & (bm > 0.0)
        cand += jnp.sum(
            jnp.where(hit, colf[:, m * CH:(m + 1) * CH], 0.0),
            axis=1, keepdims=True)
        pref += lm[:, CH - 1:CH]
    smask = jnp.where(is_diag, -3.0e38, s)
    maxv = jnp.max(smask, axis=1, keepdims=True)
    fb = jnp.min(jnp.where(smask >= maxv, colf, float(B)),
                 axis=1, keepdims=True)
    neg = jnp.where(count > 0.0, cand, fb)
    out_ref[...] = neg.astype(jnp.int32)


def _mine(t_uni, v_uni, u2, tri):
    return pl.pallas_call(
        _mine_body,
        grid=(NB,),
        in_specs=[
            pl.BlockSpec((BLK, D), lambda i: (i, 0)),
            pl.BlockSpec((B, D), lambda i: (0, 0)),
            pl.BlockSpec((BLK, 1), lambda i: (i, 0)),
            pl.BlockSpec((CH, CH), lambda i: (0, 0)),
        ],
        out_specs=pl.BlockSpec((BLK, 1), lambda i: (i, 0)),
        out_shape=jax.ShapeDtypeStruct((B, 1), jnp.int32),
    )(t_uni, v_uni, u2, tri)


def _gather(table, idx):
    info = plsc.get_sparse_core_info()
    nw = info.num_cores * info.num_subcores
    bpw = B // nw
    mesh = plsc.VectorSubcoreMesh(core_axis_name="c", subcore_axis_name="s")

    @functools.partial(
        pl.kernel, mesh=mesh,
        out_type=jax.ShapeDtypeStruct((B, D), jnp.float32),
        scratch_types=[
            pltpu.VMEM((bpw,), jnp.int32),
            pltpu.VMEM((bpw, D), jnp.float32),
            pltpu.SemaphoreType.DMA,
        ],
    )
    def k(table_hbm, idx_hbm, out_hbm, idx_v, rows_v, sem):
        wid = lax.axis_index("s") * info.num_cores + lax.axis_index("c")
        base = wid * bpw
        pltpu.sync_copy(idx_hbm.at[pl.ds(base, bpw)], idx_v)
        pltpu.async_copy(table_hbm.at[idx_v], rows_v, sem).wait()
        pltpu.sync_copy(rows_v, out_hbm.at[pl.ds(base, bpw)])

    return k(table, idx)


def _head_body(tc_ref, vc_ref, vn_ref, w1t_ref, w1v_ref, w1d_ref, b1_ref,
               w2t_ref, b2_ref, out_ref):
    i = pl.program_id(0)
    tc = tc_ref[...]
    vc = vc_ref[...]
    vn = vn_ref[...]
    w1t = w1t_ref[...]
    w1v = w1v_ref[...]
    w1d = w1d_ref[...]
    b1 = b1_ref[...]
    w2t = w2t_ref[...]
    b2 = b2_ref[...]
    a = lax.dot_general(tc, w1t, (((1,), (0,)), ((), ())),
                        preferred_element_type=jnp.float32)
    pv = lax.dot_general(vc, w1v, (((1,), (0,)), ((), ())),
                         preferred_element_type=jnp.float32)
    nv = lax.dot_general(vn, w1v, (((1,), (0,)), ((), ())),
                         preferred_element_type=jnp.float32)
    dot_pos = jnp.sum(vc * tc, axis=1, keepdims=True)
    dot_neg = jnp.sum(vn * tc, axis=1, keepdims=True)
    hp = jnp.maximum(a + pv + dot_pos * w1d + b1, 0.0)
    hn = jnp.maximum(a + nv + dot_neg * w1d + b1, 0.0)
    lp = jnp.sum(hp * w2t, axis=1, keepdims=True) + b2
    ln = jnp.sum(hn * w2t, axis=1, keepdims=True) + b2
    sp = 1.0 / (1.0 + jnp.exp(-lp))
    sn = 1.0 / (1.0 + jnp.exp(-ln))
    terms = jnp.log(sp + 1e-8) + jnp.log(1.0 - sn + 1e-8)
    p = jnp.sum(terms, axis=0, keepdims=True)

    @pl.when(i == 0)
    def _():
        out_ref[...] = jnp.zeros_like(out_ref)

    out_ref[...] += p

    @pl.when(i == NB - 1)
    def _():
        out_ref[...] = out_ref[...] * (-1.0 / (2.0 * B))


def _head(tc, vc, vn, w1t, w1v, w1d, b1r, w2t, b2r):
    return pl.pallas_call(
        _head_body,
        grid=(NB,),
        in_specs=[
            pl.BlockSpec((BLK, D), lambda i: (i, 0)),
            pl.BlockSpec((BLK, D), lambda i: (i, 0)),
            pl.BlockSpec((BLK, D), lambda i: (i, 0)),
            pl.BlockSpec((D, D), lambda i: (0, 0)),
            pl.BlockSpec((D, D), lambda i: (0, 0)),
            pl.BlockSpec((1, D), lambda i: (0, 0)),
            pl.BlockSpec((1, D), lambda i: (0, 0)),
            pl.BlockSpec((1, D), lambda i: (0, 0)),
            pl.BlockSpec((1, 1), lambda i: (0, 0)),
        ],
        out_specs=pl.BlockSpec((1, 1), lambda i: (0, 0)),
        out_shape=jax.ShapeDtypeStruct((1, 1), jnp.float32),
    )(tc, vc, vn, w1t, w1v, w1d, b1r, w2t, b2r)


def kernel(vision_embeds_cross, text_embeds_cross, vision_embeds_uni,
           text_embeds_uni, W1, b1, W2, b2):
    u = jax.random.uniform(jax.random.key(42), (B,)).reshape(B, 1)
    ar = jnp.arange(CH)
    tri = (ar[:, None] <= ar[None, :]).astype(jnp.float32)
    neg_idx = _mine(text_embeds_uni, vision_embeds_uni, u, tri).reshape(B)
    vision_neg = _gather(vision_embeds_cross, neg_idx)
    w1t = W1[0:D]
    w1v = W1[D:2 * D]
    w1d = W1[2 * D:2 * D + 1]
    b1r = b1.reshape(1, D)
    w2t = W2.reshape(1, D)
    b2r = b2.reshape(1, 1)
    out = _head(text_embeds_cross, vision_embeds_cross, vision_neg,
                w1t, w1v, w1d, b1r, w2t, b2r)
    return out[0, 0]


# speedup vs baseline: 2.9305x; 2.9305x over previous
import functools

import jax
import jax.numpy as jnp
from jax import lax
from jax.experimental import pallas as pl
from jax.experimental.pallas import tpu as pltpu
from jax.experimental.pallas import tpu_sc as plsc

MARGIN_MIN = 0.2
MARGIN_MAX = 0.5
B = 4096
D = 256
BLK = 256
NB = B // BLK
CH = 128
NCH = B // CH


def _mine_body(t_ref, v_ref, vb_ref, u_ref, tri_ref, out_ref):
    i = pl.program_id(0)
    t = t_ref[...]
    v = v_ref[...]
    vb = vb_ref[...]
    s = lax.dot_general(
        t, v, (((1,), (1,)), ((), ())),
        preferred_element_type=jnp.float32)
    diag = jnp.sum(t * vb, axis=1, keepdims=True)
    col = lax.broadcasted_iota(jnp.int32, (BLK, B), 1)
    rowi = lax.broadcasted_iota(jnp.int32, (BLK, 1), 0) + i * BLK
    bandm = (s > diag - MARGIN_MAX) & (s < diag - MARGIN_MIN)
    bf = jnp.where(bandm, 1.0, 0.0)
    cm = [jnp.sum(bf[:, m * CH:(m + 1) * CH], axis=1, keepdims=True)
          for m in range(NCH)]
    pref = [jnp.zeros((BLK, 1), jnp.float32)]
    for m in range(NCH - 1):
        pref.append(pref[m] + cm[m])
    count = pref[NCH - 1] + cm[NCH - 1]
    u = u_ref[...]
    k1 = jnp.floor(u * jnp.maximum(count, 1.0)) + 1.0
    tri = tri_ref[...]
    cand = jnp.full((BLK, 1), B, jnp.int32)
    for m in range(NCH):
        bm = bf[:, m * CH:(m + 1) * CH]
        lm = lax.dot_general(
            bm, tri, (((1,), (0,)), ((), ())),
            preferred_element_type=jnp.float32)
        sel = (pref[m] + lm) >= k1
        cand = jnp.minimum(
            cand,
            jnp.min(jnp.where(sel, col[:, m * CH:(m + 1) * CH], B),
                    axis=1, keepdims=True))
    cand = jnp.where(cand >= B, 0, cand)
    need_fb = jnp.sum(jnp.where(count <= 0.0, 1.0, 0.0)) > 0.0

    @pl.when(jnp.logical_not(need_fb))
    def _():
        out_ref[...] = cand

    @pl.when(need_fb)
    def _():
        smask = jnp.where(col == rowi, -3.0e38, s)
        maxv = jnp.max(smask, axis=1, keepdims=True)
        fb = jnp.min(jnp.where(smask >= maxv, col, B), axis=1, keepdims=True)
        out_ref[...] = jnp.where(count > 0.0, cand, fb)


def _mine(t_uni, v_uni, u2, tri):
    return pl.pallas_call(
        _mine_body,
        grid=(NB,),
        in_specs=[
            pl.BlockSpec((BLK, D), lambda i: (i, 0)),
            pl.BlockSpec((B, D), lambda i: (0, 0)),
            pl.BlockSpec((BLK, D), lambda i: (i, 0)),
            pl.BlockSpec((BLK, 1), lambda i: (i, 0)),
            pl.BlockSpec((CH, CH), lambda i: (0, 0)),
        ],
        out_specs=pl.BlockSpec((BLK, 1), lambda i: (i, 0)),
        out_shape=jax.ShapeDtypeStruct((B, 1), jnp.int32),
    )(t_uni, v_uni, v_uni, u2, tri)


def _gather(table, idx):
    info = plsc.get_sparse_core_info()
    nw = info.num_cores * info.num_subcores
    bpw = B // nw
    mesh = plsc.VectorSubcoreMesh(core_axis_name="c", subcore_axis_name="s")

    @functools.partial(
        pl.kernel, mesh=mesh,
        out_type=jax.ShapeDtypeStruct((B, D), jnp.float32),
        scratch_types=[
            pltpu.VMEM((bpw,), jnp.int32),
            pltpu.VMEM((bpw, D), jnp.float32),
            pltpu.SemaphoreType.DMA,
        ],
    )
    def k(table_hbm, idx_hbm, out_hbm, idx_v, rows_v, sem):
        wid = lax.axis_index("s") * info.num_cores + lax.axis_index("c")
        base = wid * bpw
        pltpu.sync_copy(idx_hbm.at[pl.ds(base, bpw)], idx_v)
        pltpu.async_copy(table_hbm.at[idx_v], rows_v, sem).wait()
        pltpu.sync_copy(rows_v, out_hbm.at[pl.ds(base, bpw)])

    return k(table, idx)


def _head_body(tc_ref, vc_ref, vn_ref, w1t_ref, w1v_ref, w1d_ref, b1_ref,
               w2t_ref, b2_ref, out_ref):
    i = pl.program_id(0)
    tc = tc_ref[...]
    vc = vc_ref[...]
    vn = vn_ref[...]
    w1t = w1t_ref[...]
    w1v = w1v_ref[...]
    w1d = w1d_ref[...]
    b1 = b1_ref[...]
    w2t = w2t_ref[...]
    b2 = b2_ref[...]
    a = lax.dot_general(tc, w1t, (((1,), (0,)), ((), ())),
                        preferred_element_type=jnp.float32)
    pv = lax.dot_general(vc, w1v, (((1,), (0,)), ((), ())),
                         preferred_element_type=jnp.float32)
    nv = lax.dot_general(vn, w1v, (((1,), (0,)), ((), ())),
                         preferred_element_type=jnp.float32)
    dot_pos = jnp.sum(vc * tc, axis=1, keepdims=True)
    dot_neg = jnp.sum(vn * tc, axis=1, keepdims=True)
    hp = jnp.maximum(a + pv + dot_pos * w1d + b1, 0.0)
    hn = jnp.maximum(a + nv + dot_neg * w1d + b1, 0.0)
    lp = jnp.sum(hp * w2t, axis=1, keepdims=True) + b2
    ln = jnp.sum(hn * w2t, axis=1, keepdims=True) + b2
    sp = 1.0 / (1.0 + jnp.exp(-lp))
    sn = 1.0 / (1.0 + jnp.exp(-ln))
    terms = jnp.log(sp + 1e-8) + jnp.log(1.0 - sn + 1e-8)
    p = jnp.sum(terms, axis=0, keepdims=True)

    @pl.when(i == 0)
    def _():
        out_ref[...] = jnp.zeros_like(out_ref)

    out_ref[...] += p

    @pl.when(i == NB - 1)
    def _():
        out_ref[...] = out_ref[...] * (-1.0 / (2.0 * B))


def _head(tc, vc, vn, w1t, w1v, w1d, b1r, w2t, b2r):
    return pl.pallas_call(
        _head_body,
        grid=(NB,),
        in_specs=[
            pl.BlockSpec((BLK, D), lambda i: (i, 0)),
            pl.BlockSpec((BLK, D), lambda i: (i, 0)),
            pl.BlockSpec((BLK, D), lambda i: (i, 0)),
            pl.BlockSpec((D, D), lambda i: (0, 0)),
            pl.BlockSpec((D, D), lambda i: (0, 0)),
            pl.BlockSpec((1, D), lambda i: (0, 0)),
            pl.BlockSpec((1, D), lambda i: (0, 0)),
            pl.BlockSpec((1, D), lambda i: (0, 0)),
            pl.BlockSpec((1, 1), lambda i: (0, 0)),
        ],
        out_specs=pl.BlockSpec((1, 1), lambda i: (0, 0)),
        out_shape=jax.ShapeDtypeStruct((1, 1), jnp.float32),
    )(tc, vc, vn, w1t, w1v, w1d, b1r, w2t, b2r)


def kernel(vision_embeds_cross, text_embeds_cross, vision_embeds_uni,
           text_embeds_uni, W1, b1, W2, b2):
    u = jax.random.uniform(jax.random.key(42), (B,)).reshape(B, 1)
    ar = jnp.arange(CH)
    tri = (ar[:, None] <= ar[None, :]).astype(jnp.float32)
    neg_idx = _mine(text_embeds_uni, vision_embeds_uni, u, tri).reshape(B)
    vision_neg = _gather(vision_embeds_cross, neg_idx)
    w1t = W1[0:D]
    w1v = W1[D:2 * D]
    w1d = W1[2 * D:2 * D + 1]
    b1r = b1.reshape(1, D)
    w2t = W2.reshape(1, D)
    b2r = b2.reshape(1, 1)
    out = _head(text_embeds_cross, vision_embeds_cross, vision_neg,
                w1t, w1v, w1d, b1r, w2t, b2r)
    return out[0, 0]


# speedup vs baseline: 3.7592x; 1.2828x over previous
import functools

import jax
import jax.numpy as jnp
import numpy as np
from jax import lax
from jax.experimental import pallas as pl
from jax.experimental.pallas import tpu as pltpu
from jax.experimental.pallas import tpu_sc as plsc

MARGIN_MIN = 0.2
MARGIN_MAX = 0.5
B = 4096
D = 256
BLK = 512
NB = B // BLK
CH = 128
NCH = B // CH

def _np_uniform(n, seed):
    k1 = np.uint32(seed >> 32)
    k2 = np.uint32(seed & 0xFFFFFFFF)
    x0 = np.zeros(n, np.uint32)
    x1 = np.arange(n, dtype=np.uint32)
    rot0 = (13, 15, 26, 6)
    rot1 = (17, 29, 16, 24)
    ks = (k1, k2, np.uint32(k1 ^ k2 ^ np.uint32(0x1BD11BDA)))
    sched = ((rot0, ks[1], ks[2], 1), (rot1, ks[2], ks[0], 2),
             (rot0, ks[0], ks[1], 3), (rot1, ks[1], ks[2], 4),
             (rot0, ks[2], ks[0], 5))
    with np.errstate(over='ignore'):
        x0 = (x0 + ks[0]).astype(np.uint32)
        x1 = (x1 + ks[1]).astype(np.uint32)
        for rots, a0, a1, inc in sched:
            for r in rots:
                x0 = (x0 + x1).astype(np.uint32)
                x1 = ((x1 << np.uint32(r))
                      | (x1 >> np.uint32(32 - r))).astype(np.uint32)
                x1 = (x0 ^ x1).astype(np.uint32)
            x0 = (x0 + a0).astype(np.uint32)
            x1 = (x1 + a1 + np.uint32(inc)).astype(np.uint32)
    bits = (x0 ^ x1).astype(np.uint32)
    fb = (bits >> np.uint32(9)) | np.uint32(0x3F800000)
    return fb.view(np.float32) - np.float32(1.0)


_U = _np_uniform(B, 42).reshape(B, 1)
_ar = np.arange(CH)
_TRI = (_ar[:, None] <= _ar[None, :]).astype(jnp.bfloat16)
_jb = np.arange(B)
_E = (_jb[:, None] // CH == np.arange(NCH)[None, :]).astype(jnp.bfloat16)
_LT32 = (np.arange(NCH)[:, None] < np.arange(NCH)[None, :]).astype(
    jnp.bfloat16)
_COLF = _jb[None, :].astype(np.float32)


def _mine_body(t_ref, v_ref, vb_ref, u_ref, tri_ref, e_ref, lt32_ref,
               colf_ref, out_ref):
    i = pl.program_id(0)
    t = t_ref[...]
    v = v_ref[...]
    vb = vb_ref[...]
    s = lax.dot_general(
        t, v, (((1,), (1,)), ((), ())),
        preferred_element_type=jnp.float32)
    diag = jnp.sum(t * vb, axis=1, keepdims=True)
    bandm = (s > diag - MARGIN_MAX) & (s < diag - MARGIN_MIN)
    bf = bandm.astype(jnp.bfloat16)
    c_all = lax.dot_general(bf, e_ref[...], (((1,), (0,)), ((), ())),
                            preferred_element_type=jnp.float32)
    count = jnp.sum(c_all, axis=1, keepdims=True)
    pbs = lax.dot_general(c_all.astype(jnp.bfloat16), lt32_ref[...],
                          (((1,), (0,)), ((), ())),
                          preferred_element_type=jnp.float32)
    u = u_ref[...]
    k1 = jnp.floor(u * jnp.maximum(count, 1.0)) + 1.0
    th = k1 - pbs
    tri = tri_ref[...]
    acc = jnp.zeros((BLK, CH), jnp.float32)
    for m in range(NCH):
        sl = slice(m * CH, (m + 1) * CH)
        lm = lax.dot_general(
            bf[:, sl], tri, (((1,), (0,)), ((), ())),
            preferred_element_type=jnp.float32)
        acc += jnp.where(lm < th[:, m:m + 1], 1.0, 0.0)
    cand = jnp.sum(acc, axis=1, keepdims=True)
    cand = jnp.where(cand >= float(B), 0.0, cand)
    need_fb = jnp.sum(jnp.where(count <= 0.0, 1.0, 0.0)) > 0.0

    @pl.when(jnp.logical_not(need_fb))
    def _():
        out_ref[...] = cand.astype(jnp.int32)

    @pl.when(need_fb)
    def _():
        rowf = (lax.broadcasted_iota(jnp.int32, (BLK, 1), 0)
                + i * BLK).astype(jnp.float32)
        colf = colf_ref[...]
        smask = jnp.where(colf == rowf, -3.0e38, s)
        maxv = jnp.max(smask, axis=1, keepdims=True)
        fb = jnp.min(jnp.where(smask >= maxv, colf, float(B)),
                     axis=1, keepdims=True)
        out_ref[...] = jnp.where(count > 0.0, cand, fb).astype(jnp.int32)


def _mine(t_uni, v_uni, u2):
    return pl.pallas_call(
        _mine_body,
        grid=(NB,),
        in_specs=[
            pl.BlockSpec((BLK, D), lambda i: (i, 0)),
            pl.BlockSpec((B, D), lambda i: (0, 0)),
            pl.BlockSpec((BLK, D), lambda i: (i, 0)),
            pl.BlockSpec((BLK, 1), lambda i: (i, 0)),
            pl.BlockSpec((CH, CH), lambda i: (0, 0)),
            pl.BlockSpec((B, NCH), lambda i: (0, 0)),
            pl.BlockSpec((NCH, NCH), lambda i: (0, 0)),
            pl.BlockSpec((1, B), lambda i: (0, 0)),
        ],
        out_specs=pl.BlockSpec((BLK, 1), lambda i: (i, 0)),
        out_shape=jax.ShapeDtypeStruct((B, 1), jnp.int32),
    )(t_uni, v_uni, v_uni, u2, _TRI, _E, _LT32, _COLF)


def _gather(table, idx):
    info = plsc.get_sparse_core_info()
    nw = info.num_cores * info.num_subcores
    bpw = B // nw
    mesh = plsc.VectorSubcoreMesh(core_axis_name="c", subcore_axis_name="s")

    @functools.partial(
        pl.kernel, mesh=mesh,
        out_type=jax.ShapeDtypeStruct((B, D), jnp.float32),
        scratch_types=[
            pltpu.VMEM((bpw,), jnp.int32),
            pltpu.VMEM((bpw, D), jnp.float32),
            pltpu.SemaphoreType.DMA,
        ],
    )
    def k(table_hbm, idx_hbm, out_hbm, idx_v, rows_v, sem):
        wid = lax.axis_index("s") * info.num_cores + lax.axis_index("c")
        base = wid * bpw
        pltpu.sync_copy(idx_hbm.at[pl.ds(base, bpw)], idx_v)
        pltpu.async_copy(table_hbm.at[idx_v], rows_v, sem).wait()
        pltpu.sync_copy(rows_v, out_hbm.at[pl.ds(base, bpw)])

    return k(table, idx)


def _head_body(tc_ref, vc_ref, vn_ref, w1_ref, b1_ref, w2_ref, b2_ref,
               out_ref):
    i = pl.program_id(0)
    tc = tc_ref[...]
    vc = vc_ref[...]
    vn = vn_ref[...]
    w1t = w1_ref[0:D, :]
    w1v = w1_ref[D:2 * D, :]
    w1d = w1_ref[2 * D:2 * D + 1, :]
    b1 = b1_ref[...]
    w2 = w2_ref[...]
    b2 = b2_ref[...]
    a = lax.dot_general(tc, w1t, (((1,), (0,)), ((), ())),
                        preferred_element_type=jnp.float32)
    pv = lax.dot_general(vc, w1v, (((1,), (0,)), ((), ())),
                         preferred_element_type=jnp.float32)
    nv = lax.dot_general(vn, w1v, (((1,), (0,)), ((), ())),
                         preferred_element_type=jnp.float32)
    dot_pos = jnp.sum(vc * tc, axis=1, keepdims=True)
    dot_neg = jnp.sum(vn * tc, axis=1, keepdims=True)
    hp = jnp.maximum(a + pv + dot_pos * w1d + b1, 0.0)
    hn = jnp.maximum(a + nv + dot_neg * w1d + b1, 0.0)
    lp = lax.dot_general(hp, w2, (((1,), (0,)), ((), ())),
                         preferred_element_type=jnp.float32) + b2
    ln = lax.dot_general(hn, w2, (((1,), (0,)), ((), ())),
                         preferred_element_type=jnp.float32) + b2
    sp = 1.0 / (1.0 + jnp.exp(-lp))
    sn = 1.0 / (1.0 + jnp.exp(-ln))
    terms = jnp.log(sp + 1e-8) + jnp.log(1.0 - sn + 1e-8)
    p = jnp.sum(terms, axis=0, keepdims=True)

    @pl.when(i == 0)
    def _():
        out_ref[...] = jnp.zeros_like(out_ref)

    out_ref[...] += p

    @pl.when(i == NB - 1)
    def _():
        out_ref[...] = out_ref[...] * (-1.0 / (2.0 * B))


def _head(tc, vc, vn, W1, b1r, W2, b2r):
    return pl.pallas_call(
        _head_body,
        grid=(NB,),
        in_specs=[
            pl.BlockSpec((BLK, D), lambda i: (i, 0)),
            pl.BlockSpec((BLK, D), lambda i: (i, 0)),
            pl.BlockSpec((BLK, D), lambda i: (i, 0)),
            pl.BlockSpec((2 * D + 1, D), lambda i: (0, 0)),
            pl.BlockSpec((1, D), lambda i: (0, 0)),
            pl.BlockSpec((D, 1), lambda i: (0, 0)),
            pl.BlockSpec((1, 1), lambda i: (0, 0)),
        ],
        out_specs=pl.BlockSpec((1, 1), lambda i: (0, 0)),
        out_shape=jax.ShapeDtypeStruct((1, 1), jnp.float32),
    )(tc, vc, vn, W1, b1r, W2, b2r)


def kernel(vision_embeds_cross, text_embeds_cross, vision_embeds_uni,
           text_embeds_uni, W1, b1, W2, b2):
    neg_idx = _mine(text_embeds_uni, vision_embeds_uni, _U).reshape(B)
    vision_neg = _gather(vision_embeds_cross, neg_idx)
    out = _head(text_embeds_cross, vision_embeds_cross, vision_neg,
                W1, b1.reshape(1, D), W2, b2.reshape(1, 1))
    return out[0, 0]


# speedup vs baseline: 3.8916x; 1.0352x over previous
import functools

import jax
import jax.numpy as jnp
import numpy as np
from jax import lax
from jax.experimental import pallas as pl
from jax.experimental.pallas import tpu as pltpu
from jax.experimental.pallas import tpu_sc as plsc

MARGIN_MIN = 0.2
MARGIN_MAX = 0.5
B = 4096
D = 256
BLK = 512
NB = B // BLK
CH = 128
NCH = B // CH

def _np_uniform(n, seed):
    k1 = np.uint32(seed >> 32)
    k2 = np.uint32(seed & 0xFFFFFFFF)
    x0 = np.zeros(n, np.uint32)
    x1 = np.arange(n, dtype=np.uint32)
    rot0 = (13, 15, 26, 6)
    rot1 = (17, 29, 16, 24)
    ks = (k1, k2, np.uint32(k1 ^ k2 ^ np.uint32(0x1BD11BDA)))
    sched = ((rot0, ks[1], ks[2], 1), (rot1, ks[2], ks[0], 2),
             (rot0, ks[0], ks[1], 3), (rot1, ks[1], ks[2], 4),
             (rot0, ks[2], ks[0], 5))
    with np.errstate(over='ignore'):
        x0 = (x0 + ks[0]).astype(np.uint32)
        x1 = (x1 + ks[1]).astype(np.uint32)
        for rots, a0, a1, inc in sched:
            for r in rots:
                x0 = (x0 + x1).astype(np.uint32)
                x1 = ((x1 << np.uint32(r))
                      | (x1 >> np.uint32(32 - r))).astype(np.uint32)
                x1 = (x0 ^ x1).astype(np.uint32)
            x0 = (x0 + a0).astype(np.uint32)
            x1 = (x1 + a1 + np.uint32(inc)).astype(np.uint32)
    bits = (x0 ^ x1).astype(np.uint32)
    fb = (bits >> np.uint32(9)) | np.uint32(0x3F800000)
    return fb.view(np.float32) - np.float32(1.0)


_U = _np_uniform(B, 42).reshape(B, 1)
_ar = np.arange(CH)
_TRI = (_ar[:, None] <= _ar[None, :]).astype(jnp.bfloat16)
_jb = np.arange(B)
_E = (_jb[:, None] // CH == np.arange(NCH)[None, :]).astype(jnp.bfloat16)
_LT32 = (np.arange(NCH)[:, None] < np.arange(NCH)[None, :]).astype(
    jnp.bfloat16)
_COLF = _jb[None, :].astype(np.float32)


def _mine_body(t_ref, v_ref, vb_ref, u_ref, tri_ref, e_ref, lt32_ref,
               colf_ref, out_ref):
    i = pl.program_id(0)
    t = t_ref[...]
    v = v_ref[...]
    vb = vb_ref[...]
    s = lax.dot_general(
        t, v, (((1,), (1,)), ((), ())),
        preferred_element_type=jnp.float32)
    diag = jnp.sum(t * vb, axis=1, keepdims=True)
    bandm = (s > diag - MARGIN_MAX) & (s < diag - MARGIN_MIN)
    bf = bandm.astype(jnp.bfloat16)
    c_all = lax.dot_general(bf, e_ref[...], (((1,), (0,)), ((), ())),
                            preferred_element_type=jnp.float32)
    count = jnp.sum(c_all, axis=1, keepdims=True)
    pbs = lax.dot_general(c_all.astype(jnp.bfloat16), lt32_ref[...],
                          (((1,), (0,)), ((), ())),
                          preferred_element_type=jnp.float32)
    u = u_ref[...]
    k1 = jnp.floor(u * jnp.maximum(count, 1.0)) + 1.0
    th = k1 - pbs
    tri = tri_ref[...]
    acc = jnp.zeros((BLK, CH), jnp.float32)
    for m in range(NCH):
        sl = slice(m * CH, (m + 1) * CH)
        lm = lax.dot_general(
            bf[:, sl], tri, (((1,), (0,)), ((), ())),
            preferred_element_type=jnp.float32)
        acc += jnp.where(lm < th[:, m:m + 1], 1.0, 0.0)
    cand = jnp.sum(acc, axis=1, keepdims=True)
    cand = jnp.where(cand >= float(B), 0.0, cand)
    rowf = (lax.broadcasted_iota(jnp.int32, (BLK, 1), 0)
            + i * BLK).astype(jnp.float32)
    colf = colf_ref[...]
    smask = jnp.where(colf == rowf, -3.0e38, s)
    maxv = jnp.max(smask, axis=1, keepdims=True)
    fb = jnp.min(jnp.where(smask >= maxv, colf, float(B)),
                 axis=1, keepdims=True)
    out_ref[...] = jnp.where(count > 0.0, cand, fb).astype(jnp.int32)


def _mine(t_uni, v_uni, u2):
    return pl.pallas_call(
        _mine_body,
        grid=(NB,),
        in_specs=[
            pl.BlockSpec((BLK, D), lambda i: (i, 0)),
            pl.BlockSpec((B, D), lambda i: (0, 0)),
            pl.BlockSpec((BLK, D), lambda i: (i, 0)),
            pl.BlockSpec((BLK, 1), lambda i: (i, 0)),
            pl.BlockSpec((CH, CH), lambda i: (0, 0)),
            pl.BlockSpec((B, NCH), lambda i: (0, 0)),
            pl.BlockSpec((NCH, NCH), lambda i: (0, 0)),
            pl.BlockSpec((1, B), lambda i: (0, 0)),
        ],
        out_specs=pl.BlockSpec((BLK, 1), lambda i: (i, 0)),
        out_shape=jax.ShapeDtypeStruct((B, 1), jnp.int32),
    )(t_uni, v_uni, v_uni, u2, _TRI, _E, _LT32, _COLF)


def _gather(table, idx):
    info = plsc.get_sparse_core_info()
    nw = info.num_cores * info.num_subcores
    bpw = B // nw
    mesh = plsc.VectorSubcoreMesh(core_axis_name="c", subcore_axis_name="s")

    @functools.partial(
        pl.kernel, mesh=mesh,
        out_type=jax.ShapeDtypeStruct((B, D), jnp.float32),
        scratch_types=[
            pltpu.VMEM((bpw,), jnp.int32),
            pltpu.VMEM((bpw, D), jnp.float32),
            pltpu.SemaphoreType.DMA,
        ],
    )
    def k(table_hbm, idx_hbm, out_hbm, idx_v, rows_v, sem):
        wid = lax.axis_index("s") * info.num_cores + lax.axis_index("c")
        base = wid * bpw
        pltpu.sync_copy(idx_hbm.at[pl.ds(base, bpw)], idx_v)
        pltpu.async_copy(table_hbm.at[idx_v], rows_v, sem).wait()
        pltpu.sync_copy(rows_v, out_hbm.at[pl.ds(base, bpw)])

    return k(table, idx)


def _head_body(tc_ref, vc_ref, vn_ref, w1_ref, b1_ref, w2_ref, b2_ref,
               out_ref):
    i = pl.program_id(0)
    tc = tc_ref[...]
    vc = vc_ref[...]
    vn = vn_ref[...]
    w1t = w1_ref[0:D, :]
    w1v = w1_ref[D:2 * D, :]
    w1d = w1_ref[2 * D:2 * D + 1, :]
    b1 = b1_ref[...]
    w2 = w2_ref[...]
    b2 = b2_ref[...]
    a = lax.dot_general(tc, w1t, (((1,), (0,)), ((), ())),
                        preferred_element_type=jnp.float32)
    pv = lax.dot_general(vc, w1v, (((1,), (0,)), ((), ())),
                         preferred_element_type=jnp.float32)
    nv = lax.dot_general(vn, w1v, (((1,), (0,)), ((), ())),
                         preferred_element_type=jnp.float32)
    dot_pos = jnp.sum(vc * tc, axis=1, keepdims=True)
    dot_neg = jnp.sum(vn * tc, axis=1, keepdims=True)
    hp = jnp.maximum(a + pv + dot_pos * w1d + b1, 0.0)
    hn = jnp.maximum(a + nv + dot_neg * w1d + b1, 0.0)
    lp = lax.dot_general(hp, w2, (((1,), (0,)), ((), ())),
                         preferred_element_type=jnp.float32) + b2
    ln = lax.dot_general(hn, w2, (((1,), (0,)), ((), ())),
                         preferred_element_type=jnp.float32) + b2
    sp = 1.0 / (1.0 + jnp.exp(-lp))
    sn = 1.0 / (1.0 + jnp.exp(-ln))
    terms = jnp.log(sp + 1e-8) + jnp.log(1.0 - sn + 1e-8)
    p = jnp.sum(terms, axis=0, keepdims=True)

    @pl.when(i == 0)
    def _():
        out_ref[...] = jnp.zeros_like(out_ref)

    out_ref[...] += p

    @pl.when(i == NB - 1)
    def _():
        out_ref[...] = out_ref[...] * (-1.0 / (2.0 * B))


def _head(tc, vc, vn, W1, b1r, W2, b2r):
    return pl.pallas_call(
        _head_body,
        grid=(NB,),
        in_specs=[
            pl.BlockSpec((BLK, D), lambda i: (i, 0)),
            pl.BlockSpec((BLK, D), lambda i: (i, 0)),
            pl.BlockSpec((BLK, D), lambda i: (i, 0)),
            pl.BlockSpec((2 * D + 1, D), lambda i: (0, 0)),
            pl.BlockSpec((1, D), lambda i: (0, 0)),
            pl.BlockSpec((D, 1), lambda i: (0, 0)),
            pl.BlockSpec((1, 1), lambda i: (0, 0)),
        ],
        out_specs=pl.BlockSpec((1, 1), lambda i: (0, 0)),
        out_shape=jax.ShapeDtypeStruct((1, 1), jnp.float32),
    )(tc, vc, vn, W1, b1r, W2, b2r)


def kernel(vision_embeds_cross, text_embeds_cross, vision_embeds_uni,
           text_embeds_uni, W1, b1, W2, b2):
    neg_idx = _mine(text_embeds_uni, vision_embeds_uni, _U).reshape(B)
    vision_neg = _gather(vision_embeds_cross, neg_idx)
    out = _head(text_embeds_cross, vision_embeds_cross, vision_neg,
                W1, b1.reshape(1, D), W2, b2.reshape(1, 1))
    return out[0, 0]
